# Initial kernel scaffold; baseline (speedup 1.0000x reference)
import jax
import jax.numpy as jnp
from jax.experimental import pallas as pl


def kernel(srcs, pos, reference_points, spatial_shapes, level_start_index, padding_mask, params, ln_w, ln_b):
    raise NotImplementedError("write your pallas kernel here")


# speedup vs baseline: 1.0010x; 1.0010x over previous
import functools

import jax
import jax.numpy as jnp
from jax.experimental import pallas as pl

FEAT_NUM = 2
SLICE_NUM = 4
N_HEADS = 8
N_POINTS = 4
D_MODEL = 256
D_HEAD = D_MODEL // N_HEADS
SPATIAL = [(100, 100), (50, 50), (25, 25), (13, 13)]
LSI = [0, 10000, 12500, 13125]
LEN_IN = 13294
BATCH = 2

BQ = 512


def _sample_level(v, loc, H, W):
    x = loc[..., 0] * W - 0.5
    y = loc[..., 1] * H - 0.5
    x0 = jnp.floor(x)
    y0 = jnp.floor(y)
    x0i = x0.astype(jnp.int32)
    y0i = y0.astype(jnp.int32)
    x1i = x0i + 1
    y1i = y0i + 1
    wx1 = x - x0
    wx0 = 1.0 - wx1
    wy1 = y - y0
    wy0 = 1.0 - wy1
    flat = v.reshape(v.shape[0], H * W, v.shape[-1])

    def gather(ix, iy):
        valid = ((ix >= 0) & (ix < W) & (iy >= 0) & (iy < H)).astype(v.dtype)
        ixc = jnp.clip(ix, 0, W - 1)
        iyc = jnp.clip(iy, 0, H - 1)
        idx = iyc * W + ixc
        g = jnp.take_along_axis(flat, idx[..., None], axis=1)
        return g * valid[..., None]

    out = (gather(x0i, y0i) * (wx0 * wy0)[..., None]
           + gather(x1i, y0i) * (wx1 * wy0)[..., None]
           + gather(x0i, y1i) * (wx0 * wy1)[..., None]
           + gather(x1i, y1i) * (wx1 * wy1)[..., None])
    return out


def _attn_body(p, query, ref_pts):
    N, Lq, C = query.shape
    value = query @ p["value_w"].T + p["value_b"]
    value = value.reshape(N, LEN_IN, N_HEADS, D_HEAD)
    so = (query @ p["so_w"].T + p["so_b"]).reshape(N, Lq, N_HEADS, SLICE_NUM, N_POINTS, 2)
    aw = (query @ p["aw_w"].T + p["aw_b"]).reshape(N, Lq, N_HEADS, SLICE_NUM * N_POINTS)
    aw = jax.nn.softmax(aw, axis=-1).reshape(N, Lq, N_HEADS, SLICE_NUM, N_POINTS)
    offset_norm = jnp.array([[w, h] for (h, w) in SPATIAL], dtype=query.dtype)
    loc = ref_pts[:, :, None, :, None, :] + so / offset_norm[None, None, None, :, None, :]
    level_out = []
    for l, (H, W) in enumerate(SPATIAL):
        s = LSI[l]
        v = value[:, s:s + H * W].reshape(N, H, W, N_HEADS, D_HEAD)
        v = v.transpose(0, 3, 1, 2, 4).reshape(N * N_HEADS, H, W, D_HEAD)
        ll = loc[:, :, :, l].transpose(0, 2, 1, 3, 4).reshape(N * N_HEADS, Lq * N_POINTS, 2)
        samp = _sample_level(v, ll, H, W).reshape(N, N_HEADS, Lq, N_POINTS, D_HEAD)
        level_out.append(samp)
    samp = jnp.stack(level_out, axis=3)
    wgt = aw.transpose(0, 2, 1, 3, 4)
    out = (samp * wgt[..., None]).sum(axis=(3, 4)).transpose(0, 2, 1, 3).reshape(N, Lq, C)
    return out


def _finish_body(src_ref, attn_ref, ow_ref, ob_ref, lw_ref, lb_ref, o_ref):
    src = src_ref[...]
    attn = attn_ref[...]
    y = src + jax.lax.dot_general(
        attn, ow_ref[...], (((1,), (1,)), ((), ())),
        preferred_element_type=jnp.float32) + ob_ref[...]
    mu = jnp.mean(y, axis=-1, keepdims=True)
    var = jnp.mean((y - mu) ** 2, axis=-1, keepdims=True)
    o_ref[...] = (y - mu) * jax.lax.rsqrt(var + 1e-5) * lw_ref[...] + lb_ref[...]


def _finish(src, attn, out_w, out_b, ln_w, ln_b):
    R = src.shape[0]
    grid = (R // BQ,)
    return pl.pallas_call(
        _finish_body,
        grid=grid,
        in_specs=[
            pl.BlockSpec((BQ, D_MODEL), lambda i: (i, 0)),
            pl.BlockSpec((BQ, D_MODEL), lambda i: (i, 0)),
            pl.BlockSpec((D_MODEL, D_MODEL), lambda i: (0, 0)),
            pl.BlockSpec((D_MODEL,), lambda i: (0,)),
            pl.BlockSpec((D_MODEL,), lambda i: (0,)),
            pl.BlockSpec((D_MODEL,), lambda i: (0,)),
        ],
        out_specs=pl.BlockSpec((BQ, D_MODEL), lambda i: (i, 0)),
        out_shape=jax.ShapeDtypeStruct((R, D_MODEL), jnp.float32),
    )(src, attn, out_w, out_b, ln_w, ln_b)


def kernel(srcs, pos, reference_points, spatial_shapes, level_start_index,
           padding_mask, params, ln_w, ln_b):
    del spatial_shapes, level_start_index, padding_mask
    outs = []
    for lvl in range(FEAT_NUM):
        src = srcs[lvl] + pos[lvl]
        p = params[lvl]
        attn = _attn_body(p, src, reference_points[lvl])
        R = BATCH * LEN_IN
        RP = ((R + BQ - 1) // BQ) * BQ
        src_f = src.reshape(R, D_MODEL)
        attn_f = attn.reshape(R, D_MODEL)
        pad = RP - R
        src_f = jnp.pad(src_f, ((0, pad), (0, 0)))
        attn_f = jnp.pad(attn_f, ((0, pad), (0, 0)))
        o = _finish(src_f, attn_f, p["out_w"], p["out_b"], ln_w, ln_b)
        outs.append(o[:R].reshape(BATCH, LEN_IN, D_MODEL))
    return jnp.stack(outs, axis=0)


# speedup vs baseline: 34.3676x; 34.3319x over previous
import functools

import numpy as np
import jax
import jax.numpy as jnp
from jax import lax
from jax.experimental import pallas as pl
from jax.experimental.pallas import tpu as pltpu
from jax.experimental.pallas import tpu_sc as plsc

FEAT_NUM = 2
SLICE_NUM = 4
N_HEADS = 8
N_POINTS = 4
D_MODEL = 256
D_HEAD = D_MODEL // N_HEADS
SPATIAL = [(100, 100), (50, 50), (25, 25), (13, 13)]
LSI = [0, 10000, 12500, 13125]
LEN_IN = 13294
BATCH = 2

NW = FEAT_NUM * BATCH * N_HEADS
QC = 512
NQP = 13312
NA_CHUNKS = NQP // QC
CSC = 16
SC_CHUNKS = NQP // CSC
IDX_ROWS = CSC * 64 // 128
BQ = 512

def _lane_const(vals, dtype):
    lvl = lax.broadcasted_iota(jnp.int32, (1, 16), 1) // N_POINTS
    out = jnp.full((1, 16), vals[-1], dtype)
    for l in range(len(vals) - 2, -1, -1):
        out = jnp.where(lvl == l, jnp.asarray(vals[l], dtype), out)
    return out


def _a_body(src_ref, pos_ref, ref_ref, wcat_ref, bcat_ref,
            vt_ref, idx_ref, wgt_ref):
    w = pl.program_id(0)
    src = src_ref[0, 0] + pos_ref[0, 0]
    proj = lax.dot_general(src.astype(jnp.bfloat16), wcat_ref[0, 0],
                           (((1,), (0,)), ((), ())),
                           preferred_element_type=jnp.float32)
    proj = proj + bcat_ref[0, 0, 0:1, :]
    vt_ref[...] = proj[:, 0:32]
    sox = proj[:, 32:48]
    soy = proj[:, 48:64]
    aw = jax.nn.softmax(proj[:, 64:80], axis=-1)
    rx = ref_ref[0, 0][:, 0:16]
    ry = ref_ref[0, 0][:, 16:32]
    wvals = [w for (h, w) in SPATIAL]
    hvals = [h for (h, w) in SPATIAL]
    wf = _lane_const(wvals, jnp.float32)
    hf = _lane_const(hvals, jnp.float32)
    wi = _lane_const(wvals, jnp.int32)
    hi = _lane_const(hvals, jnp.int32)
    li = _lane_const(LSI, jnp.int32)
    x = rx * wf + sox - 0.5
    y = ry * hf + soy - 0.5
    x0f = jnp.floor(x)
    y0f = jnp.floor(y)
    fx = x - x0f
    fy = y - y0f
    x0 = x0f.astype(jnp.int32)
    y0 = y0f.astype(jnp.int32)
    x1 = x0 + 1
    y1 = y0 + 1
    vx0 = (x0 >= 0) & (x0 < wi)
    vx1 = (x1 >= 0) & (x1 < wi)
    vy0 = (y0 >= 0) & (y0 < hi)
    vy1 = (y1 >= 0) & (y1 < hi)
    cx0 = jnp.clip(x0, 0, wi - 1)
    cx1 = jnp.clip(x1, 0, wi - 1)
    cy0 = jnp.clip(y0, 0, hi - 1)
    cy1 = jnp.clip(y1, 0, hi - 1)
    base = li + w * NQP
    r0 = base + cy0 * wi
    r1 = base + cy1 * wi
    gx0 = 1.0 - fx
    gy0 = 1.0 - fy
    w00 = aw * gx0 * gy0 * (vx0 & vy0).astype(jnp.float32)
    w01 = aw * fx * gy0 * (vx1 & vy0).astype(jnp.float32)
    w10 = aw * gx0 * fy * (vx0 & vy1).astype(jnp.float32)
    w11 = aw * fx * fy * (vx1 & vy1).astype(jnp.float32)
    idx_ref[...] = jnp.concatenate([r0 + cx0, r0 + cx1, r1 + cx0, r1 + cx1],
                                   axis=-1)
    wgt_ref[...] = jnp.concatenate([w00, w01, w10, w11], axis=-1)


def _phase_a(src_pad, pos_pad, refxy, wcat, bcat, interpret=False):
    grid = (NW, NA_CHUNKS)
    return pl.pallas_call(
        _a_body,
        grid=grid,
        in_specs=[
            pl.BlockSpec((1, 1, QC, D_MODEL),
                         lambda w, qi: (w // 16, (w // 8) % 2, qi, 0)),
            pl.BlockSpec((1, 1, QC, D_MODEL),
                         lambda w, qi: (w // 16, (w // 8) % 2, qi, 0)),
            pl.BlockSpec((1, 1, QC, 32),
                         lambda w, qi: (w // 16, (w // 8) % 2, qi, 0)),
            pl.BlockSpec((1, 1, D_MODEL, 128),
                         lambda w, qi: (w // 16, w % 8, 0, 0)),
            pl.BlockSpec((1, 1, 8, 128),
                         lambda w, qi: (w // 16, w % 8, 0, 0)),
        ],
        out_specs=[
            pl.BlockSpec((QC, 32), lambda w, qi: (w * NA_CHUNKS + qi, 0)),
            pl.BlockSpec((QC, 64), lambda w, qi: (w * NA_CHUNKS + qi, 0)),
            pl.BlockSpec((QC, 64), lambda w, qi: (w * NA_CHUNKS + qi, 0)),
        ],
        out_shape=[
            jax.ShapeDtypeStruct((NW * NQP, 32), jnp.float32),
            jax.ShapeDtypeStruct((NW * NQP, 64), jnp.int32),
            jax.ShapeDtypeStruct((NW * NQP, 64), jnp.float32),
        ],
        interpret=interpret,
    )(src_pad, pos_pad, refxy, wcat, bcat)


def _sc_gather_mac(vt_flat, idx3, wgt2):
    mesh = plsc.VectorSubcoreMesh(core_axis_name="c", subcore_axis_name="s")

    @functools.partial(
        pl.kernel,
        mesh=mesh,
        compiler_params=pltpu.CompilerParams(use_tc_tiling_on_sc=False),
        out_type=jax.ShapeDtypeStruct((NW * NQP, 32), jnp.float32),
        scratch_types=[
            pltpu.VMEM((IDX_ROWS, 128), jnp.int32),
            pltpu.VMEM((CSC * 4, 16), jnp.float32),
            pltpu.VMEM((CSC * 64, 32), jnp.float32),
            pltpu.VMEM((CSC, 32), jnp.float32),
            pltpu.SemaphoreType.DMA,
        ],
    )
    def k(vt_hbm, idx_hbm, wgt_hbm, out_hbm, idx_v, wgt_v, rows_v, out_v, sem):
        nc = plsc.get_sparse_core_info().num_cores
        wid = lax.axis_index("s") * nc + lax.axis_index("c")

        def chunk_body(ch, carry):
            base = wid * SC_CHUNKS + ch
            pltpu.sync_copy(idx_hbm.at[base], idx_v)
            pltpu.sync_copy(wgt_hbm.at[base], wgt_v)
            copies = [
                pltpu.async_copy(vt_hbm.at[idx_v.at[j]],
                                 rows_v.at[pl.ds(j * 128, 128)], sem)
                for j in range(IDX_ROWS)
            ]
            for c in copies:
                c.wait()

            def q_body(q, c2):
                k0 = q * 64
                acc0 = jnp.zeros((16,), jnp.float32)
                acc1 = jnp.zeros((16,), jnp.float32)
                for t in range(4):
                    wv = wgt_v[q * 4 + t, :]
                    for e in range(16):
                        jj = t * 16 + e
                        wq = wv[e]
                        acc0 = acc0 + wq * rows_v[k0 + jj, pl.ds(0, 16)]
                        acc1 = acc1 + wq * rows_v[k0 + jj, pl.ds(16, 16)]
                out_v[q, pl.ds(0, 16)] = acc0
                out_v[q, pl.ds(16, 16)] = acc1
                return c2

            lax.fori_loop(0, CSC, q_body, 0)
            pltpu.sync_copy(out_v,
                            out_hbm.at[pl.ds(wid * NQP + ch * CSC, CSC)])
            return carry

        lax.fori_loop(0, SC_CHUNKS, chunk_body, 0)

    return k(vt_flat, idx3, wgt2)


def _finish_body(srcs_ref, pos_ref, attn_ref, ow_ref, ob_ref, lw_ref, lb_ref,
                 o_ref):
    src = srcs_ref[...] + pos_ref[...]
    y = src + lax.dot_general(
        attn_ref[...], ow_ref[...], (((1,), (1,)), ((), ())),
        preferred_element_type=jnp.float32) + ob_ref[...]
    mu = jnp.mean(y, axis=-1, keepdims=True)
    var = jnp.mean((y - mu) ** 2, axis=-1, keepdims=True)
    o_ref[...] = (y - mu) * lax.rsqrt(var + 1e-5) * lw_ref[...] + lb_ref[...]


def _finish(srcs_f, pos_f, attn, out_w, out_b, ln_w, ln_b, interpret=False):
    R = srcs_f.shape[0]
    grid = (R // BQ,)
    return pl.pallas_call(
        _finish_body,
        grid=grid,
        in_specs=[
            pl.BlockSpec((BQ, D_MODEL), lambda i: (i, 0)),
            pl.BlockSpec((BQ, D_MODEL), lambda i: (i, 0)),
            pl.BlockSpec((BQ, D_MODEL), lambda i: (i, 0)),
            pl.BlockSpec((D_MODEL, D_MODEL), lambda i: (0, 0)),
            pl.BlockSpec((D_MODEL,), lambda i: (0,)),
            pl.BlockSpec((D_MODEL,), lambda i: (0,)),
            pl.BlockSpec((D_MODEL,), lambda i: (0,)),
        ],
        out_specs=pl.BlockSpec((BQ, D_MODEL), lambda i: (i, 0)),
        out_shape=jax.ShapeDtypeStruct((R, D_MODEL), jnp.float32),
        interpret=interpret,
    )(srcs_f, pos_f, attn, out_w, out_b, ln_w, ln_b)


def _prep_weights(params):
    wcats, bcats = [], []
    for f in range(FEAT_NUM):
        p = params[f]
        vw = p["value_w"].reshape(N_HEADS, D_HEAD, D_MODEL)
        sow = p["so_w"].reshape(N_HEADS, SLICE_NUM, N_POINTS, 2, D_MODEL)
        soxw = sow[..., 0, :].reshape(N_HEADS, 16, D_MODEL)
        soyw = sow[..., 1, :].reshape(N_HEADS, 16, D_MODEL)
        aww = p["aw_w"].reshape(N_HEADS, 16, D_MODEL)
        wc = jnp.concatenate([vw, soxw, soyw, aww], axis=1)
        wc = jnp.pad(wc, ((0, 0), (0, 48), (0, 0)))
        wcats.append(wc.transpose(0, 2, 1))
        vb = p["value_b"].reshape(N_HEADS, D_HEAD)
        sob = p["so_b"].reshape(N_HEADS, SLICE_NUM, N_POINTS, 2)
        soxb = sob[..., 0].reshape(N_HEADS, 16)
        soyb = sob[..., 1].reshape(N_HEADS, 16)
        awb = p["aw_b"].reshape(N_HEADS, 16)
        bc = jnp.concatenate([vb, soxb, soyb, awb], axis=1)
        bc = jnp.pad(bc, ((0, 0), (0, 48)))
        bcats.append(jnp.broadcast_to(bc[:, None, :], (N_HEADS, 8, 128)))
    wcat = jnp.stack(wcats).astype(jnp.bfloat16)
    bcat = jnp.stack(bcats)
    return wcat, bcat


def kernel(srcs, pos, reference_points, spatial_shapes, level_start_index,
           padding_mask, params, ln_w, ln_b):
    del spatial_shapes, level_start_index, padding_mask
    pad_q = NQP - LEN_IN
    src_pad = jnp.pad(srcs, ((0, 0), (0, 0), (0, pad_q), (0, 0)))
    pos_pad = jnp.pad(pos, ((0, 0), (0, 0), (0, pad_q), (0, 0)))
    rx = jnp.repeat(reference_points[..., 0], N_POINTS, axis=-1)
    ry = jnp.repeat(reference_points[..., 1], N_POINTS, axis=-1)
    refxy = jnp.pad(jnp.concatenate([rx, ry], axis=-1),
                    ((0, 0), (0, 0), (0, pad_q), (0, 0)))
    wcat, bcat = _prep_weights(params)

    vt_flat, idxo, wgto = _phase_a(src_pad, pos_pad, refxy, wcat, bcat)
    idx3 = idxo.reshape(NW * SC_CHUNKS, IDX_ROWS, 128)
    wgt2 = wgto.reshape(NW * SC_CHUNKS, CSC * 4, 16)

    attn_flat = _sc_gather_mac(vt_flat, idx3, wgt2)

    attn = attn_flat.reshape(FEAT_NUM, BATCH, N_HEADS, NQP, 32)
    attn = attn[:, :, :, :LEN_IN].transpose(0, 1, 3, 2, 4)
    attn = attn.reshape(FEAT_NUM, BATCH, LEN_IN, D_MODEL)

    R = BATCH * LEN_IN
    RP = ((R + BQ - 1) // BQ) * BQ
    outs = []
    for lvl in range(FEAT_NUM):
        p = params[lvl]
        srcs_f = jnp.pad(srcs[lvl].reshape(R, D_MODEL), ((0, RP - R), (0, 0)))
        pos_f = jnp.pad(pos[lvl].reshape(R, D_MODEL), ((0, RP - R), (0, 0)))
        attn_f = jnp.pad(attn[lvl].reshape(R, D_MODEL), ((0, RP - R), (0, 0)))
        o = _finish(srcs_f, pos_f, attn_f, p["out_w"], p["out_b"], ln_w, ln_b)
        outs.append(o[:R].reshape(BATCH, LEN_IN, D_MODEL))
    return jnp.stack(outs, axis=0)


# speedup vs baseline: 43.1429x; 1.2553x over previous
import functools

import numpy as np
import jax
import jax.numpy as jnp
from jax import lax
from jax.experimental import pallas as pl
from jax.experimental.pallas import tpu as pltpu
from jax.experimental.pallas import tpu_sc as plsc

FEAT_NUM = 2
SLICE_NUM = 4
N_HEADS = 8
N_POINTS = 4
D_MODEL = 256
D_HEAD = D_MODEL // N_HEADS
SPATIAL = [(100, 100), (50, 50), (25, 25), (13, 13)]
LSI = [0, 10000, 12500, 13125]
LEN_IN = 13294
BATCH = 2

NW = FEAT_NUM * BATCH * N_HEADS
QC = 512
NQP = 13312
NA_CHUNKS = NQP // QC
CSC = 16
SC_CHUNKS = NQP // CSC
IDX_ROWS = CSC * 64 // 128
BQ = 512

def _lane_const(vals, dtype):
    lvl = lax.broadcasted_iota(jnp.int32, (1, 16), 1) // N_POINTS
    out = jnp.full((1, 16), vals[-1], dtype)
    for l in range(len(vals) - 2, -1, -1):
        out = jnp.where(lvl == l, jnp.asarray(vals[l], dtype), out)
    return out


def _a_body(src_ref, pos_ref, ref_ref, wcat_ref, bcat_ref,
            vt_ref, idx_ref, wgt_ref):
    w = pl.program_id(0)
    src = src_ref[0, 0] + pos_ref[0, 0]
    proj = lax.dot_general(src.astype(jnp.bfloat16), wcat_ref[0, 0],
                           (((1,), (0,)), ((), ())),
                           preferred_element_type=jnp.float32)
    proj = proj + bcat_ref[0, 0, 0:1, :]
    vt_ref[...] = proj[:, 0:32]
    sox = proj[:, 32:48]
    soy = proj[:, 48:64]
    aw = jax.nn.softmax(proj[:, 64:80], axis=-1)
    rx = ref_ref[0, 0][:, 0:16]
    ry = ref_ref[0, 0][:, 16:32]
    wvals = [w for (h, w) in SPATIAL]
    hvals = [h for (h, w) in SPATIAL]
    wf = _lane_const(wvals, jnp.float32)
    hf = _lane_const(hvals, jnp.float32)
    wi = _lane_const(wvals, jnp.int32)
    hi = _lane_const(hvals, jnp.int32)
    li = _lane_const(LSI, jnp.int32)
    x = rx * wf + sox - 0.5
    y = ry * hf + soy - 0.5
    x0f = jnp.floor(x)
    y0f = jnp.floor(y)
    fx = x - x0f
    fy = y - y0f
    x0 = x0f.astype(jnp.int32)
    y0 = y0f.astype(jnp.int32)
    x1 = x0 + 1
    y1 = y0 + 1
    vx0 = (x0 >= 0) & (x0 < wi)
    vx1 = (x1 >= 0) & (x1 < wi)
    vy0 = (y0 >= 0) & (y0 < hi)
    vy1 = (y1 >= 0) & (y1 < hi)
    cx0 = jnp.clip(x0, 0, wi - 1)
    cx1 = jnp.clip(x1, 0, wi - 1)
    cy0 = jnp.clip(y0, 0, hi - 1)
    cy1 = jnp.clip(y1, 0, hi - 1)
    base = li + w * NQP
    r0 = base + cy0 * wi
    r1 = base + cy1 * wi
    gx0 = 1.0 - fx
    gy0 = 1.0 - fy
    w00 = aw * gx0 * gy0 * (vx0 & vy0).astype(jnp.float32)
    w01 = aw * fx * gy0 * (vx1 & vy0).astype(jnp.float32)
    w10 = aw * gx0 * fy * (vx0 & vy1).astype(jnp.float32)
    w11 = aw * fx * fy * (vx1 & vy1).astype(jnp.float32)
    idx_ref[...] = jnp.concatenate([r0 + cx0, r0 + cx1, r1 + cx0, r1 + cx1],
                                   axis=-1)
    wgt_ref[...] = jnp.concatenate([w00, w01, w10, w11], axis=-1)


def _phase_a(src_pad, pos_pad, refxy, wcat, bcat, interpret=False):
    grid = (NW, NA_CHUNKS)
    return pl.pallas_call(
        _a_body,
        grid=grid,
        in_specs=[
            pl.BlockSpec((1, 1, QC, D_MODEL),
                         lambda w, qi: (w // 16, (w // 8) % 2, qi, 0)),
            pl.BlockSpec((1, 1, QC, D_MODEL),
                         lambda w, qi: (w // 16, (w // 8) % 2, qi, 0)),
            pl.BlockSpec((1, 1, QC, 32),
                         lambda w, qi: (w // 16, (w // 8) % 2, qi, 0)),
            pl.BlockSpec((1, 1, D_MODEL, 128),
                         lambda w, qi: (w // 16, w % 8, 0, 0)),
            pl.BlockSpec((1, 1, 8, 128),
                         lambda w, qi: (w // 16, w % 8, 0, 0)),
        ],
        out_specs=[
            pl.BlockSpec((QC, 32), lambda w, qi: (w * NA_CHUNKS + qi, 0)),
            pl.BlockSpec((QC, 64), lambda w, qi: (w * NA_CHUNKS + qi, 0)),
            pl.BlockSpec((QC, 64), lambda w, qi: (w * NA_CHUNKS + qi, 0)),
        ],
        out_shape=[
            jax.ShapeDtypeStruct((NW * NQP, 32), jnp.float32),
            jax.ShapeDtypeStruct((NW * NQP, 64), jnp.int32),
            jax.ShapeDtypeStruct((NW * NQP, 64), jnp.float32),
        ],
        interpret=interpret,
    )(src_pad, pos_pad, refxy, wcat, bcat)


GRP = 4
NG = SC_CHUNKS // GRP


def _sc_gather_mac(vt_flat, idx3, wgt3):
    mesh = plsc.VectorSubcoreMesh(core_axis_name="c", subcore_axis_name="s")

    @functools.partial(
        pl.kernel,
        mesh=mesh,
        compiler_params=pltpu.CompilerParams(use_tc_tiling_on_sc=False),
        out_type=jax.ShapeDtypeStruct((NW * NQP, 32), jnp.float32),
        scratch_types=[
            pltpu.VMEM((GRP * 8, 128), jnp.int32),
            pltpu.VMEM((GRP * 8, 128), jnp.int32),
            pltpu.VMEM((GRP * CSC * 4, 16), jnp.float32),
            pltpu.VMEM((GRP * CSC * 4, 16), jnp.float32),
            pltpu.VMEM((CSC * 64, 32), jnp.float32),
            pltpu.VMEM((CSC * 64, 32), jnp.float32),
            pltpu.VMEM((GRP * CSC, 32), jnp.float32),
            pltpu.SemaphoreType.DMA,
            pltpu.SemaphoreType.DMA,
            pltpu.SemaphoreType.DMA,
            pltpu.SemaphoreType.DMA,
        ],
    )
    def k(vt_hbm, idx_hbm, wgt_hbm, out_hbm,
          ig0, ig1, wg0, wg1, r0, r1, outg, sg0, sg1, sr0, sr1):
        igs, wgs, rs = [ig0, ig1], [wg0, wg1], [r0, r1]
        sgs, srs = [sg0, sg1], [sr0, sr1]
        nc = plsc.get_sparse_core_info().num_cores
        wid = lax.axis_index("s") * nc + lax.axis_index("c")
        gbase = wid * NG

        def fire(ig, ch, q):
            return [
                pltpu.async_copy(vt_hbm.at[ig.at[ch * 8 + j]],
                                 rs[q].at[pl.ds(j * 128, 128)], srs[q])
                for j in range(IDX_ROWS)
            ]

        def do_group(g, p):
            @pl.when(g + 1 < NG)
            def _():
                pltpu.async_copy(idx_hbm.at[gbase + g + 1], igs[1 - p],
                                 sgs[1 - p])
                pltpu.async_copy(wgt_hbm.at[gbase + g + 1], wgs[1 - p],
                                 sgs[1 - p])

            pending = fire(igs[p], 0, 0)
            for ch in range(GRP):
                q = ch % 2
                nxt = []
                if ch + 1 < GRP:
                    nxt = fire(igs[p], ch + 1, 1 - q)
                for c in pending:
                    c.wait()
                pending = nxt

                def q_body(qq, c2):
                    acc0 = jnp.zeros((16,), jnp.float32)
                    acc1 = jnp.zeros((16,), jnp.float32)
                    for t in range(4):
                        wv = wgs[p][(ch * CSC + qq) * 4 + t, :]
                        for e in range(16):
                            jj = qq * 64 + t * 16 + e
                            wq = wv[e]
                            acc0 = acc0 + wq * rs[q][jj, pl.ds(0, 16)]
                            acc1 = acc1 + wq * rs[q][jj, pl.ds(16, 16)]
                    outg[ch * CSC + qq, pl.ds(0, 16)] = acc0
                    outg[ch * CSC + qq, pl.ds(16, 16)] = acc1
                    return c2

                lax.fori_loop(0, CSC, q_body, 0)
            pltpu.sync_copy(
                outg, out_hbm.at[pl.ds(wid * NQP + g * GRP * CSC, GRP * CSC)])

        pltpu.sync_copy(idx_hbm.at[gbase], igs[0])
        pltpu.sync_copy(wgt_hbm.at[gbase], wgs[0])

        def pair_body(go, carry):
            for b in range(2):
                g = go * 2 + b
                p = b
                @pl.when(g > 0)
                def _():
                    pltpu.make_async_copy(idx_hbm.at[gbase + g], igs[p],
                                          sgs[p]).wait()
                    pltpu.make_async_copy(wgt_hbm.at[gbase + g], wgs[p],
                                          sgs[p]).wait()

                do_group(g, p)
            return carry

        lax.fori_loop(0, NG // 2, pair_body, 0)

    return k(vt_flat, idx3, wgt3)


def _finish_body(srcs_ref, pos_ref, attn_ref, ow_ref, ob_ref, lw_ref, lb_ref,
                 o_ref):
    src = srcs_ref[...] + pos_ref[...]
    y = src + lax.dot_general(
        attn_ref[...], ow_ref[...], (((1,), (1,)), ((), ())),
        preferred_element_type=jnp.float32) + ob_ref[...]
    mu = jnp.mean(y, axis=-1, keepdims=True)
    var = jnp.mean((y - mu) ** 2, axis=-1, keepdims=True)
    o_ref[...] = (y - mu) * lax.rsqrt(var + 1e-5) * lw_ref[...] + lb_ref[...]


def _finish(srcs_f, pos_f, attn, out_w, out_b, ln_w, ln_b, interpret=False):
    R = srcs_f.shape[0]
    grid = (R // BQ,)
    return pl.pallas_call(
        _finish_body,
        grid=grid,
        in_specs=[
            pl.BlockSpec((BQ, D_MODEL), lambda i: (i, 0)),
            pl.BlockSpec((BQ, D_MODEL), lambda i: (i, 0)),
            pl.BlockSpec((BQ, D_MODEL), lambda i: (i, 0)),
            pl.BlockSpec((D_MODEL, D_MODEL), lambda i: (0, 0)),
            pl.BlockSpec((D_MODEL,), lambda i: (0,)),
            pl.BlockSpec((D_MODEL,), lambda i: (0,)),
            pl.BlockSpec((D_MODEL,), lambda i: (0,)),
        ],
        out_specs=pl.BlockSpec((BQ, D_MODEL), lambda i: (i, 0)),
        out_shape=jax.ShapeDtypeStruct((R, D_MODEL), jnp.float32),
        interpret=interpret,
    )(srcs_f, pos_f, attn, out_w, out_b, ln_w, ln_b)


def _prep_weights(params):
    wcats, bcats = [], []
    for f in range(FEAT_NUM):
        p = params[f]
        vw = p["value_w"].reshape(N_HEADS, D_HEAD, D_MODEL)
        sow = p["so_w"].reshape(N_HEADS, SLICE_NUM, N_POINTS, 2, D_MODEL)
        soxw = sow[..., 0, :].reshape(N_HEADS, 16, D_MODEL)
        soyw = sow[..., 1, :].reshape(N_HEADS, 16, D_MODEL)
        aww = p["aw_w"].reshape(N_HEADS, 16, D_MODEL)
        wc = jnp.concatenate([vw, soxw, soyw, aww], axis=1)
        wc = jnp.pad(wc, ((0, 0), (0, 48), (0, 0)))
        wcats.append(wc.transpose(0, 2, 1))
        vb = p["value_b"].reshape(N_HEADS, D_HEAD)
        sob = p["so_b"].reshape(N_HEADS, SLICE_NUM, N_POINTS, 2)
        soxb = sob[..., 0].reshape(N_HEADS, 16)
        soyb = sob[..., 1].reshape(N_HEADS, 16)
        awb = p["aw_b"].reshape(N_HEADS, 16)
        bc = jnp.concatenate([vb, soxb, soyb, awb], axis=1)
        bc = jnp.pad(bc, ((0, 0), (0, 48)))
        bcats.append(jnp.broadcast_to(bc[:, None, :], (N_HEADS, 8, 128)))
    wcat = jnp.stack(wcats).astype(jnp.bfloat16)
    bcat = jnp.stack(bcats)
    return wcat, bcat


def kernel(srcs, pos, reference_points, spatial_shapes, level_start_index,
           padding_mask, params, ln_w, ln_b):
    del spatial_shapes, level_start_index, padding_mask
    pad_q = NQP - LEN_IN
    src_pad = jnp.pad(srcs, ((0, 0), (0, 0), (0, pad_q), (0, 0)))
    pos_pad = jnp.pad(pos, ((0, 0), (0, 0), (0, pad_q), (0, 0)))
    rx = jnp.repeat(reference_points[..., 0], N_POINTS, axis=-1)
    ry = jnp.repeat(reference_points[..., 1], N_POINTS, axis=-1)
    refxy = jnp.pad(jnp.concatenate([rx, ry], axis=-1),
                    ((0, 0), (0, 0), (0, pad_q), (0, 0)))
    wcat, bcat = _prep_weights(params)

    vt_flat, idxo, wgto = _phase_a(src_pad, pos_pad, refxy, wcat, bcat)
    idx3 = idxo.reshape(NW * NG, GRP * 8, 128)
    wgt3 = wgto.reshape(NW * NG, GRP * CSC * 4, 16)

    attn_flat = _sc_gather_mac(vt_flat, idx3, wgt3)

    attn = attn_flat.reshape(FEAT_NUM, BATCH, N_HEADS, NQP, 32)
    attn = attn[:, :, :, :LEN_IN].transpose(0, 1, 3, 2, 4)
    attn = attn.reshape(FEAT_NUM, BATCH, LEN_IN, D_MODEL)

    R = BATCH * LEN_IN
    RP = ((R + BQ - 1) // BQ) * BQ
    outs = []
    for lvl in range(FEAT_NUM):
        p = params[lvl]
        srcs_f = jnp.pad(srcs[lvl].reshape(R, D_MODEL), ((0, RP - R), (0, 0)))
        pos_f = jnp.pad(pos[lvl].reshape(R, D_MODEL), ((0, RP - R), (0, 0)))
        attn_f = jnp.pad(attn[lvl].reshape(R, D_MODEL), ((0, RP - R), (0, 0)))
        o = _finish(srcs_f, pos_f, attn_f, p["out_w"], p["out_b"], ln_w, ln_b)
        outs.append(o[:R].reshape(BATCH, LEN_IN, D_MODEL))
    return jnp.stack(outs, axis=0)


# speedup vs baseline: 78.3384x; 1.8158x over previous
import functools

import jax
import jax.numpy as jnp
from jax import lax
from jax.experimental import pallas as pl
from jax.experimental.pallas import tpu as pltpu
from jax.experimental.pallas import tpu_sc as plsc

FEAT_NUM = 2
SLICE_NUM = 4
N_HEADS = 8
N_POINTS = 4
D_MODEL = 256
D_HEAD = D_MODEL // N_HEADS
SPATIAL = [(100, 100), (50, 50), (25, 25), (13, 13)]
LSI = [0, 10000, 12500, 13125]
LEN_IN = 13294
BATCH = 2

NFB = FEAT_NUM * BATCH
NW = NFB * N_HEADS
QC = 512
NQP = 13312
NA_CHUNKS = NQP // QC
NPROJ = 5 * 128
CSC = 16
GRP = 4
GQ = GRP * CSC
NG = NQP // GQ
BQ = 512


def _lane_const(vals, dtype):
    lvl = (lax.broadcasted_iota(jnp.int32, (1, 128), 1) % 16) // N_POINTS
    out = jnp.full((1, 128), vals[-1], dtype)
    for l in range(len(vals) - 2, -1, -1):
        out = jnp.where(lvl == l, jnp.asarray(vals[l], dtype), out)
    return out


def _a_body(src_ref, pos_ref, ref_ref, wcat_ref, bcat_ref,
            vt_ref, idx_ref, wgt_ref):
    fb = pl.program_id(0)
    src = src_ref[0, 0] + pos_ref[0, 0]
    proj = lax.dot_general(src.astype(jnp.bfloat16), wcat_ref[0],
                           (((1,), (0,)), ((), ())),
                           preferred_element_type=jnp.float32)
    proj = proj + bcat_ref[0, 0:1, :]
    vt_ref[0] = proj[:, 0:256]
    sox = proj[:, 256:384]
    soy = proj[:, 384:512]
    e = jnp.exp(proj[:, 512:640])
    jj = lax.broadcasted_iota(jnp.int32, (128, 8), 0)
    hh = lax.broadcasted_iota(jnp.int32, (128, 8), 1)
    msk = (jj // 16 == hh).astype(jnp.float32)
    denom = lax.dot_general(e, msk, (((1,), (0,)), ((), ())),
                            preferred_element_type=jnp.float32)
    dlane = lax.dot_general(denom, msk, (((1,), (1,)), ((), ())),
                            preferred_element_type=jnp.float32)
    aw = e / dlane
    rxy = ref_ref[0, 0]
    rx = jnp.concatenate([rxy[:, 0:16]] * N_HEADS, axis=-1)
    ry = jnp.concatenate([rxy[:, 16:32]] * N_HEADS, axis=-1)
    wvals = [w for (h, w) in SPATIAL]
    hvals = [h for (h, w) in SPATIAL]
    wf = _lane_const(wvals, jnp.float32)
    hf = _lane_const(hvals, jnp.float32)
    wi = _lane_const(wvals, jnp.int32)
    hi = _lane_const(hvals, jnp.int32)
    li = _lane_const(LSI, jnp.int32)
    hl = lax.broadcasted_iota(jnp.int32, (1, 128), 1) // 16
    x = rx * wf + sox - 0.5
    y = ry * hf + soy - 0.5
    x0f = jnp.floor(x)
    y0f = jnp.floor(y)
    fx = x - x0f
    fy = y - y0f
    x0 = x0f.astype(jnp.int32)
    y0 = y0f.astype(jnp.int32)
    x1 = x0 + 1
    y1 = y0 + 1
    vx0 = (x0 >= 0) & (x0 < wi)
    vx1 = (x1 >= 0) & (x1 < wi)
    vy0 = (y0 >= 0) & (y0 < hi)
    vy1 = (y1 >= 0) & (y1 < hi)
    cx0 = jnp.clip(x0, 0, wi - 1)
    cx1 = jnp.clip(x1, 0, wi - 1)
    cy0 = jnp.clip(y0, 0, hi - 1)
    cy1 = jnp.clip(y1, 0, hi - 1)
    base = li + fb * NQP
    r0 = base + cy0 * wi
    r1 = base + cy1 * wi
    idx_ref[0, 0] = (r0 + cx0) * 8 + hl
    idx_ref[0, 1] = (r0 + cx1) * 8 + hl
    idx_ref[0, 2] = (r1 + cx0) * 8 + hl
    idx_ref[0, 3] = (r1 + cx1) * 8 + hl
    gx0 = 1.0 - fx
    gy0 = 1.0 - fy
    wgt_ref[0, 0] = aw * gx0 * gy0 * (vx0 & vy0).astype(jnp.float32)
    wgt_ref[0, 1] = aw * fx * gy0 * (vx1 & vy0).astype(jnp.float32)
    wgt_ref[0, 2] = aw * gx0 * fy * (vx0 & vy1).astype(jnp.float32)
    wgt_ref[0, 3] = aw * fx * fy * (vx1 & vy1).astype(jnp.float32)


def _phase_a(src_pad, pos_pad, refxy, wcat, bcat, interpret=False):
    grid = (NFB, NA_CHUNKS)
    return pl.pallas_call(
        _a_body,
        grid=grid,
        in_specs=[
            pl.BlockSpec((1, 1, QC, D_MODEL),
                         lambda fb, qi: (fb // 2, fb % 2, qi, 0)),
            pl.BlockSpec((1, 1, QC, D_MODEL),
                         lambda fb, qi: (fb // 2, fb % 2, qi, 0)),
            pl.BlockSpec((1, 1, QC, 32),
                         lambda fb, qi: (fb // 2, fb % 2, qi, 0)),
            pl.BlockSpec((1, D_MODEL, NPROJ), lambda fb, qi: (fb // 2, 0, 0)),
            pl.BlockSpec((1, 8, NPROJ), lambda fb, qi: (fb // 2, 0, 0)),
        ],
        out_specs=[
            pl.BlockSpec((1, QC, D_MODEL), lambda fb, qi: (fb, qi, 0)),
            pl.BlockSpec((1, 4, QC, 128), lambda fb, qi: (fb, 0, qi, 0)),
            pl.BlockSpec((1, 4, QC, 128), lambda fb, qi: (fb, 0, qi, 0)),
        ],
        out_shape=[
            jax.ShapeDtypeStruct((NFB, NQP, D_MODEL), jnp.float32),
            jax.ShapeDtypeStruct((NFB, 4, NQP, 128), jnp.int32),
            jax.ShapeDtypeStruct((NFB, 4, NQP, 128), jnp.float32),
        ],
        interpret=interpret,
    )(src_pad, pos_pad, refxy, wcat, bcat)


def _sc_gather_mac(vt_flat, idx4, wgt4):
    mesh = plsc.VectorSubcoreMesh(core_axis_name="c", subcore_axis_name="s")

    @functools.partial(
        pl.kernel,
        mesh=mesh,
        compiler_params=pltpu.CompilerParams(use_tc_tiling_on_sc=False),
        out_type=jax.ShapeDtypeStruct((NFB, NQP, D_MODEL), jnp.float32),
        scratch_types=[
            pltpu.VMEM((GQ, 64), jnp.int32),
            pltpu.VMEM((GQ, 64), jnp.int32),
            pltpu.VMEM((GQ, 64), jnp.float32),
            pltpu.VMEM((GQ, 64), jnp.float32),
            pltpu.VMEM((CSC * 64, 32), jnp.float32),
            pltpu.VMEM((CSC * 64, 32), jnp.float32),
            pltpu.VMEM((GQ, 32), jnp.float32),
            pltpu.SemaphoreType.DMA,
            pltpu.SemaphoreType.DMA,
            pltpu.SemaphoreType.DMA,
            pltpu.SemaphoreType.DMA,
        ],
    )
    def k(vt_hbm, idx_hbm, wgt_hbm, out_hbm,
          ig0, ig1, wg0, wg1, r0, r1, outg, sg0, sg1, sr0, sr1):
        igs, wgs, rs = [ig0, ig1], [wg0, wg1], [r0, r1]
        sgs, srs = [sg0, sg1], [sr0, sr1]
        nc = plsc.get_sparse_core_info().num_cores
        wid = lax.axis_index("s") * nc + lax.axis_index("c")
        fb = wid // N_HEADS
        hd = wid % N_HEADS

        def group_copies(g, b, make_only):
            q0 = g * GQ
            f = pltpu.make_async_copy if make_only else pltpu.async_copy
            cps = []
            for c in range(4):
                cps.append(f(idx_hbm.at[fb, c, pl.ds(q0, GQ),
                                        pl.ds(hd * 16, 16)],
                             igs[b].at[:, pl.ds(c * 16, 16)], sgs[b]))
                cps.append(f(wgt_hbm.at[fb, c, pl.ds(q0, GQ),
                                        pl.ds(hd * 16, 16)],
                             wgs[b].at[:, pl.ds(c * 16, 16)], sgs[b]))
            return cps

        def fire(ig, ch, q):
            return [
                pltpu.async_copy(vt_hbm.at[ig.at[ch * CSC + lq]],
                                 rs[q].at[pl.ds(lq * 64, 64)], srs[q])
                for lq in range(CSC)
            ]

        def do_group(g, p):
            @pl.when(g + 1 < NG)
            def _():
                group_copies(g + 1, 1 - p, False)

            pending = fire(igs[p], 0, 0)
            for ch in range(GRP):
                q = ch % 2
                nxt = fire(igs[p], ch + 1, 1 - q) if ch + 1 < GRP else []
                for cp in pending:
                    cp.wait()
                pending = nxt

                def q_body(qq, c2):
                    acc0 = jnp.zeros((16,), jnp.float32)
                    acc1 = jnp.zeros((16,), jnp.float32)
                    for t in range(4):
                        wv = wgs[p][ch * CSC + qq, pl.ds(t * 16, 16)]
                        for e2 in range(16):
                            jx = qq * 64 + t * 16 + e2
                            wq = wv[e2]
                            acc0 = acc0 + wq * rs[q][jx, pl.ds(0, 16)]
                            acc1 = acc1 + wq * rs[q][jx, pl.ds(16, 16)]
                    outg[ch * CSC + qq, pl.ds(0, 16)] = acc0
                    outg[ch * CSC + qq, pl.ds(16, 16)] = acc1
                    return c2

                lax.fori_loop(0, CSC, q_body, 0)
            pltpu.sync_copy(outg,
                            out_hbm.at[fb, pl.ds(g * GQ, GQ),
                                       pl.ds(hd * 32, 32)])

        for cp in group_copies(0, 0, False):
            cp.wait()

        def pair_body(go, carry):
            for b in range(2):
                g = go * 2 + b

                @pl.when(g > 0)
                def _():
                    for cp in group_copies(g, b, True):
                        cp.wait()

                do_group(g, b)
            return carry

        lax.fori_loop(0, NG // 2, pair_body, 0)

    return k(vt_flat, idx4, wgt4)


def _finish_body(srcs_ref, pos_ref, attn_ref, ow_ref, ob_ref, lw_ref, lb_ref,
                 o_ref):
    src = srcs_ref[0, 0] + pos_ref[0, 0]
    y = src + lax.dot_general(
        attn_ref[0], ow_ref[0], (((1,), (1,)), ((), ())),
        preferred_element_type=jnp.float32) + ob_ref[0, 0:1, :]
    mu = jnp.mean(y, axis=-1, keepdims=True)
    var = jnp.mean((y - mu) ** 2, axis=-1, keepdims=True)
    o_ref[0] = (y - mu) * lax.rsqrt(var + 1e-5) * lw_ref[...] + lb_ref[...]


def _finish(src_pad, pos_pad, attn, ow_s, ob_s, ln_w, ln_b, interpret=False):
    grid = (NFB, NA_CHUNKS)
    return pl.pallas_call(
        _finish_body,
        grid=grid,
        in_specs=[
            pl.BlockSpec((1, 1, BQ, D_MODEL),
                         lambda fb, qi: (fb // 2, fb % 2, qi, 0)),
            pl.BlockSpec((1, 1, BQ, D_MODEL),
                         lambda fb, qi: (fb // 2, fb % 2, qi, 0)),
            pl.BlockSpec((1, BQ, D_MODEL), lambda fb, qi: (fb, qi, 0)),
            pl.BlockSpec((1, D_MODEL, D_MODEL), lambda fb, qi: (fb // 2, 0, 0)),
            pl.BlockSpec((1, 8, D_MODEL), lambda fb, qi: (fb // 2, 0, 0)),
            pl.BlockSpec((D_MODEL,), lambda fb, qi: (0,)),
            pl.BlockSpec((D_MODEL,), lambda fb, qi: (0,)),
        ],
        out_specs=pl.BlockSpec((1, BQ, D_MODEL), lambda fb, qi: (fb, qi, 0)),
        out_shape=jax.ShapeDtypeStruct((NFB, LEN_IN, D_MODEL), jnp.float32),
        interpret=interpret,
    )(src_pad, pos_pad, attn, ow_s, ob_s, ln_w, ln_b)


def _prep_weights(params):
    wcats, bcats, ows, obs = [], [], [], []
    for f in range(FEAT_NUM):
        p = params[f]
        sow = p["so_w"].reshape(N_HEADS, 16, 2, D_MODEL)
        wc = jnp.concatenate([
            p["value_w"],
            sow[:, :, 0, :].reshape(128, D_MODEL),
            sow[:, :, 1, :].reshape(128, D_MODEL),
            p["aw_w"],
        ], axis=0)
        wcats.append(wc.T)
        sob = p["so_b"].reshape(N_HEADS, 16, 2)
        bc = jnp.concatenate([
            p["value_b"], sob[:, :, 0].reshape(128),
            sob[:, :, 1].reshape(128), p["aw_b"]], axis=0)
        bcats.append(jnp.broadcast_to(bc[None, :], (8, NPROJ)))
        ows.append(p["out_w"])
        obs.append(jnp.broadcast_to(p["out_b"][None, :], (8, D_MODEL)))
    wcat = jnp.stack(wcats).astype(jnp.bfloat16)
    bcat = jnp.stack(bcats)
    return wcat, bcat, jnp.stack(ows), jnp.stack(obs)


def kernel(srcs, pos, reference_points, spatial_shapes, level_start_index,
           padding_mask, params, ln_w, ln_b):
    del spatial_shapes, level_start_index, padding_mask
    pad_q = NQP - LEN_IN
    src_pad = jnp.pad(srcs, ((0, 0), (0, 0), (0, pad_q), (0, 0)))
    pos_pad = jnp.pad(pos, ((0, 0), (0, 0), (0, pad_q), (0, 0)))
    rx = jnp.repeat(reference_points[..., 0], N_POINTS, axis=-1)
    ry = jnp.repeat(reference_points[..., 1], N_POINTS, axis=-1)
    refxy = jnp.pad(jnp.concatenate([rx, ry], axis=-1),
                    ((0, 0), (0, 0), (0, pad_q), (0, 0)))
    wcat, bcat, ow_s, ob_s = _prep_weights(params)

    vt, idx4, wgt4 = _phase_a(src_pad, pos_pad, refxy, wcat, bcat)
    vt_flat = vt.reshape(NFB * NQP * 8, 32)

    attn = _sc_gather_mac(vt_flat, idx4, wgt4)

    out = _finish(src_pad, pos_pad, attn, ow_s, ob_s, ln_w, ln_b)
    return out.reshape(FEAT_NUM, BATCH, LEN_IN, D_MODEL)


# speedup vs baseline: 83.1817x; 1.0618x over previous
import functools

import jax
import jax.numpy as jnp
from jax import lax
from jax.experimental import pallas as pl
from jax.experimental.pallas import tpu as pltpu
from jax.experimental.pallas import tpu_sc as plsc

FEAT_NUM = 2
SLICE_NUM = 4
N_HEADS = 8
N_POINTS = 4
D_MODEL = 256
D_HEAD = D_MODEL // N_HEADS
SPATIAL = [(100, 100), (50, 50), (25, 25), (13, 13)]
LSI = [0, 10000, 12500, 13125]
LEN_IN = 13294
BATCH = 2

NFB = FEAT_NUM * BATCH
NW = NFB * N_HEADS
QC = 512
NQP = 13312
NA_CHUNKS = NQP // QC
NPROJ = 5 * 128
CSC = 16
GRP = 4
GQ = GRP * CSC
NG = NQP // GQ
BQ = 512


def _lane_const(vals, dtype):
    lvl = (lax.broadcasted_iota(jnp.int32, (1, 128), 1) % 16) // N_POINTS
    out = jnp.full((1, 128), vals[-1], dtype)
    for l in range(len(vals) - 2, -1, -1):
        out = jnp.where(lvl == l, jnp.asarray(vals[l], dtype), out)
    return out


def _a_body(src_ref, pos_ref, ref_ref, wcat_ref, bcat_ref,
            vt_ref, idx_ref, wgt_ref):
    fb = pl.program_id(0)
    src = src_ref[0, 0] + pos_ref[0, 0]
    proj = lax.dot_general(src.astype(jnp.bfloat16), wcat_ref[0],
                           (((1,), (0,)), ((), ())),
                           preferred_element_type=jnp.float32)
    proj = proj + bcat_ref[0, 0:1, :]
    vt_ref[0] = proj[:, 0:256].astype(jnp.bfloat16)
    sox = proj[:, 256:384]
    soy = proj[:, 384:512]
    e = jnp.exp(proj[:, 512:640])
    jj = lax.broadcasted_iota(jnp.int32, (128, 8), 0)
    hh = lax.broadcasted_iota(jnp.int32, (128, 8), 1)
    msk = (jj // 16 == hh).astype(jnp.float32)
    denom = lax.dot_general(e, msk, (((1,), (0,)), ((), ())),
                            preferred_element_type=jnp.float32)
    dlane = lax.dot_general(denom, msk, (((1,), (1,)), ((), ())),
                            preferred_element_type=jnp.float32)
    aw = e / dlane
    rxy = ref_ref[0, 0]
    rx = jnp.concatenate([rxy[:, 0:16]] * N_HEADS, axis=-1)
    ry = jnp.concatenate([rxy[:, 16:32]] * N_HEADS, axis=-1)
    wvals = [w for (h, w) in SPATIAL]
    hvals = [h for (h, w) in SPATIAL]
    wf = _lane_const(wvals, jnp.float32)
    hf = _lane_const(hvals, jnp.float32)
    wi = _lane_const(wvals, jnp.int32)
    hi = _lane_const(hvals, jnp.int32)
    li = _lane_const(LSI, jnp.int32)
    hl = lax.broadcasted_iota(jnp.int32, (1, 128), 1) // 16
    x = rx * wf + sox - 0.5
    y = ry * hf + soy - 0.5
    x0f = jnp.floor(x)
    y0f = jnp.floor(y)
    fx = x - x0f
    fy = y - y0f
    x0 = x0f.astype(jnp.int32)
    y0 = y0f.astype(jnp.int32)
    x1 = x0 + 1
    y1 = y0 + 1
    vx0 = (x0 >= 0) & (x0 < wi)
    vx1 = (x1 >= 0) & (x1 < wi)
    vy0 = (y0 >= 0) & (y0 < hi)
    vy1 = (y1 >= 0) & (y1 < hi)
    cx0 = jnp.clip(x0, 0, wi - 1)
    cx1 = jnp.clip(x1, 0, wi - 1)
    cy0 = jnp.clip(y0, 0, hi - 1)
    cy1 = jnp.clip(y1, 0, hi - 1)
    base = li + fb * NQP
    r0 = base + cy0 * wi
    r1 = base + cy1 * wi
    idx_ref[0, 0] = (r0 + cx0) * 8 + hl
    idx_ref[0, 1] = (r0 + cx1) * 8 + hl
    idx_ref[0, 2] = (r1 + cx0) * 8 + hl
    idx_ref[0, 3] = (r1 + cx1) * 8 + hl
    gx0 = 1.0 - fx
    gy0 = 1.0 - fy
    wgt_ref[0, 0] = aw * gx0 * gy0 * (vx0 & vy0).astype(jnp.float32)
    wgt_ref[0, 1] = aw * fx * gy0 * (vx1 & vy0).astype(jnp.float32)
    wgt_ref[0, 2] = aw * gx0 * fy * (vx0 & vy1).astype(jnp.float32)
    wgt_ref[0, 3] = aw * fx * fy * (vx1 & vy1).astype(jnp.float32)


def _phase_a(src_pad, pos_pad, refxy, wcat, bcat, interpret=False):
    grid = (NFB, NA_CHUNKS)
    return pl.pallas_call(
        _a_body,
        grid=grid,
        in_specs=[
            pl.BlockSpec((1, 1, QC, D_MODEL),
                         lambda fb, qi: (fb // 2, fb % 2, qi, 0)),
            pl.BlockSpec((1, 1, QC, D_MODEL),
                         lambda fb, qi: (fb // 2, fb % 2, qi, 0)),
            pl.BlockSpec((1, 1, QC, 32),
                         lambda fb, qi: (fb // 2, fb % 2, qi, 0)),
            pl.BlockSpec((1, D_MODEL, NPROJ), lambda fb, qi: (fb // 2, 0, 0)),
            pl.BlockSpec((1, 8, NPROJ), lambda fb, qi: (fb // 2, 0, 0)),
        ],
        out_specs=[
            pl.BlockSpec((1, QC, D_MODEL), lambda fb, qi: (fb, qi, 0)),
            pl.BlockSpec((1, 4, QC, 128), lambda fb, qi: (fb, 0, qi, 0)),
            pl.BlockSpec((1, 4, QC, 128), lambda fb, qi: (fb, 0, qi, 0)),
        ],
        out_shape=[
            jax.ShapeDtypeStruct((NFB, NQP, D_MODEL), jnp.bfloat16),
            jax.ShapeDtypeStruct((NFB, 4, NQP, 128), jnp.int32),
            jax.ShapeDtypeStruct((NFB, 4, NQP, 128), jnp.float32),
        ],
        interpret=interpret,
    )(src_pad, pos_pad, refxy, wcat, bcat)


def _sc_gather_mac(vt_flat, idx4, wgt4):
    mesh = plsc.VectorSubcoreMesh(core_axis_name="c", subcore_axis_name="s")

    @functools.partial(
        pl.kernel,
        mesh=mesh,
        compiler_params=pltpu.CompilerParams(use_tc_tiling_on_sc=False,
                                             needs_layout_passes=False),
        out_type=jax.ShapeDtypeStruct((NFB, NQP, D_MODEL), jnp.float32),
        scratch_types=[
            pltpu.VMEM((GQ, 64), jnp.int32),
            pltpu.VMEM((GQ, 64), jnp.int32),
            pltpu.VMEM((GQ, 64), jnp.float32),
            pltpu.VMEM((GQ, 64), jnp.float32),
            pltpu.VMEM((CSC * 64, 32), jnp.bfloat16),
            pltpu.VMEM((CSC * 64, 32), jnp.bfloat16),
            pltpu.VMEM((GQ, 32), jnp.float32),
            pltpu.SemaphoreType.DMA,
            pltpu.SemaphoreType.DMA,
            pltpu.SemaphoreType.DMA,
            pltpu.SemaphoreType.DMA,
        ],
    )
    def k(vt_hbm, idx_hbm, wgt_hbm, out_hbm,
          ig0, ig1, wg0, wg1, r0, r1, outg, sg0, sg1, sr0, sr1):
        igs, wgs, rs = [ig0, ig1], [wg0, wg1], [r0, r1]
        sgs, srs = [sg0, sg1], [sr0, sr1]
        nc = plsc.get_sparse_core_info().num_cores
        wid = lax.axis_index("s") * nc + lax.axis_index("c")
        fb = wid // N_HEADS
        hd = wid % N_HEADS

        def group_copies(g, b, make_only):
            q0 = g * GQ
            f = pltpu.make_async_copy if make_only else pltpu.async_copy
            cps = []
            for c in range(4):
                cps.append(f(idx_hbm.at[fb, c, pl.ds(q0, GQ),
                                        pl.ds(hd * 16, 16)],
                             igs[b].at[:, pl.ds(c * 16, 16)], sgs[b]))
                cps.append(f(wgt_hbm.at[fb, c, pl.ds(q0, GQ),
                                        pl.ds(hd * 16, 16)],
                             wgs[b].at[:, pl.ds(c * 16, 16)], sgs[b]))
            return cps

        def fire(ig, ch, q):
            return [
                pltpu.async_copy(vt_hbm.at[ig.at[ch * CSC + lq]],
                                 rs[q].at[pl.ds(lq * 64, 64)], srs[q])
                for lq in range(CSC)
            ]

        def do_group(g, p):
            @pl.when(g + 1 < NG)
            def _():
                group_copies(g + 1, 1 - p, False)

            pending = fire(igs[p], 0, 0)
            for ch in range(GRP):
                q = ch % 2
                nxt = fire(igs[p], ch + 1, 1 - q) if ch + 1 < GRP else []
                for cp in pending:
                    cp.wait()
                pending = nxt

                def q_body(qq, c2):
                    acc0 = jnp.zeros((16,), jnp.float32)
                    acc1 = jnp.zeros((16,), jnp.float32)
                    for t in range(4):
                        wv = wgs[p][ch * CSC + qq, pl.ds(t * 16, 16)]
                        for e2 in range(16):
                            jx = qq * 64 + t * 16 + e2
                            wq = wv[e2]
                            lo, hi = plsc.unpack(
                                rs[q][jx, :],
                                format=plsc.PackFormat.INTERLEAVED)
                            acc0 = acc0 + wq * lo
                            acc1 = acc1 + wq * hi
                    outg[ch * CSC + qq, pl.ds(0, 16)] = acc0
                    outg[ch * CSC + qq, pl.ds(16, 16)] = acc1
                    return c2

                lax.fori_loop(0, CSC, q_body, 0)
            pltpu.sync_copy(outg,
                            out_hbm.at[fb, pl.ds(g * GQ, GQ),
                                       pl.ds(hd * 32, 32)])

        for cp in group_copies(0, 0, False):
            cp.wait()

        def pair_body(go, carry):
            for b in range(2):
                g = go * 2 + b

                @pl.when(g > 0)
                def _():
                    for cp in group_copies(g, b, True):
                        cp.wait()

                do_group(g, b)
            return carry

        lax.fori_loop(0, NG // 2, pair_body, 0)

    return k(vt_flat, idx4, wgt4)


def _finish_body(srcs_ref, pos_ref, attn_ref, ow_ref, ob_ref, lw_ref, lb_ref,
                 o_ref):
    src = srcs_ref[0, 0] + pos_ref[0, 0]
    y = src + lax.dot_general(
        attn_ref[0], ow_ref[0], (((1,), (1,)), ((), ())),
        preferred_element_type=jnp.float32) + ob_ref[0, 0:1, :]
    mu = jnp.mean(y, axis=-1, keepdims=True)
    var = jnp.mean((y - mu) ** 2, axis=-1, keepdims=True)
    o_ref[0] = (y - mu) * lax.rsqrt(var + 1e-5) * lw_ref[...] + lb_ref[...]


def _finish(src_pad, pos_pad, attn, ow_s, ob_s, ln_w, ln_b, interpret=False):
    grid = (NFB, NA_CHUNKS)
    return pl.pallas_call(
        _finish_body,
        grid=grid,
        in_specs=[
            pl.BlockSpec((1, 1, BQ, D_MODEL),
                         lambda fb, qi: (fb // 2, fb % 2, qi, 0)),
            pl.BlockSpec((1, 1, BQ, D_MODEL),
                         lambda fb, qi: (fb // 2, fb % 2, qi, 0)),
            pl.BlockSpec((1, BQ, D_MODEL), lambda fb, qi: (fb, qi, 0)),
            pl.BlockSpec((1, D_MODEL, D_MODEL), lambda fb, qi: (fb // 2, 0, 0)),
            pl.BlockSpec((1, 8, D_MODEL), lambda fb, qi: (fb // 2, 0, 0)),
            pl.BlockSpec((D_MODEL,), lambda fb, qi: (0,)),
            pl.BlockSpec((D_MODEL,), lambda fb, qi: (0,)),
        ],
        out_specs=pl.BlockSpec((1, BQ, D_MODEL), lambda fb, qi: (fb, qi, 0)),
        out_shape=jax.ShapeDtypeStruct((NFB, LEN_IN, D_MODEL), jnp.float32),
        interpret=interpret,
    )(src_pad, pos_pad, attn, ow_s, ob_s, ln_w, ln_b)


def _prep_weights(params):
    wcats, bcats, ows, obs = [], [], [], []
    for f in range(FEAT_NUM):
        p = params[f]
        sow = p["so_w"].reshape(N_HEADS, 16, 2, D_MODEL)
        vw = (p["value_w"].reshape(N_HEADS, 2, 16, D_MODEL)
              .transpose(0, 2, 1, 3).reshape(D_MODEL, D_MODEL))
        vb = (p["value_b"].reshape(N_HEADS, 2, 16)
              .transpose(0, 2, 1).reshape(D_MODEL))
        wc = jnp.concatenate([
            vw,
            sow[:, :, 0, :].reshape(128, D_MODEL),
            sow[:, :, 1, :].reshape(128, D_MODEL),
            p["aw_w"],
        ], axis=0)
        wcats.append(wc.T)
        sob = p["so_b"].reshape(N_HEADS, 16, 2)
        bc = jnp.concatenate([
            vb, sob[:, :, 0].reshape(128),
            sob[:, :, 1].reshape(128), p["aw_b"]], axis=0)
        bcats.append(jnp.broadcast_to(bc[None, :], (8, NPROJ)))
        ows.append(p["out_w"])
        obs.append(jnp.broadcast_to(p["out_b"][None, :], (8, D_MODEL)))
    wcat = jnp.stack(wcats).astype(jnp.bfloat16)
    bcat = jnp.stack(bcats)
    return wcat, bcat, jnp.stack(ows), jnp.stack(obs)


def kernel(srcs, pos, reference_points, spatial_shapes, level_start_index,
           padding_mask, params, ln_w, ln_b):
    del spatial_shapes, level_start_index, padding_mask
    pad_q = NQP - LEN_IN
    src_pad = jnp.pad(srcs, ((0, 0), (0, 0), (0, pad_q), (0, 0)))
    pos_pad = jnp.pad(pos, ((0, 0), (0, 0), (0, pad_q), (0, 0)))
    rx = jnp.repeat(reference_points[..., 0], N_POINTS, axis=-1)
    ry = jnp.repeat(reference_points[..., 1], N_POINTS, axis=-1)
    refxy = jnp.pad(jnp.concatenate([rx, ry], axis=-1),
                    ((0, 0), (0, 0), (0, pad_q), (0, 0)))
    wcat, bcat, ow_s, ob_s = _prep_weights(params)

    vt, idx4, wgt4 = _phase_a(src_pad, pos_pad, refxy, wcat, bcat)
    vt_flat = vt.reshape(NFB * NQP * 8, 32)

    attn = _sc_gather_mac(vt_flat, idx4, wgt4)

    out = _finish(src_pad, pos_pad, attn, ow_s, ob_s, ln_w, ln_b)
    return out.reshape(FEAT_NUM, BATCH, LEN_IN, D_MODEL)


# speedup vs baseline: 83.3705x; 1.0023x over previous
import functools

import jax
import jax.numpy as jnp
from jax import lax
from jax.experimental import pallas as pl
from jax.experimental.pallas import tpu as pltpu
from jax.experimental.pallas import tpu_sc as plsc

FEAT_NUM = 2
SLICE_NUM = 4
N_HEADS = 8
N_POINTS = 4
D_MODEL = 256
D_HEAD = D_MODEL // N_HEADS
SPATIAL = [(100, 100), (50, 50), (25, 25), (13, 13)]
LSI = [0, 10000, 12500, 13125]
LEN_IN = 13294
BATCH = 2

NFB = FEAT_NUM * BATCH
NW = NFB * N_HEADS
QC = 512
NQP = 13312
NA_CHUNKS = NQP // QC
NPROJ = 5 * 128
CSC = 16
GRP = 4
GQ = GRP * CSC
NG = NQP // GQ
BQ = 512


def _lane_const(vals, dtype):
    lvl = (lax.broadcasted_iota(jnp.int32, (1, 128), 1) % 16) // N_POINTS
    out = jnp.full((1, 128), vals[-1], dtype)
    for l in range(len(vals) - 2, -1, -1):
        out = jnp.where(lvl == l, jnp.asarray(vals[l], dtype), out)
    return out


def _a_body(src_ref, pos_ref, ref_ref, wcat_ref, bcat_ref,
            vt_ref, idx_ref, wgt_ref):
    fb = pl.program_id(0)
    src = src_ref[0, 0] + pos_ref[0, 0]
    proj = lax.dot_general(src.astype(jnp.bfloat16), wcat_ref[0],
                           (((1,), (0,)), ((), ())),
                           preferred_element_type=jnp.float32)
    proj = proj + bcat_ref[0, 0:1, :]
    vt_ref[0] = proj[:, 0:256].astype(jnp.bfloat16)
    sox = proj[:, 256:384]
    soy = proj[:, 384:512]
    e = jnp.exp(proj[:, 512:640])
    jj = lax.broadcasted_iota(jnp.int32, (128, 8), 0)
    hh = lax.broadcasted_iota(jnp.int32, (128, 8), 1)
    msk = (jj // 16 == hh).astype(jnp.float32)
    denom = lax.dot_general(e, msk, (((1,), (0,)), ((), ())),
                            preferred_element_type=jnp.float32)
    dlane = lax.dot_general(denom, msk, (((1,), (1,)), ((), ())),
                            preferred_element_type=jnp.float32)
    aw = e / dlane
    rxy = ref_ref[0, 0]
    rx = jnp.concatenate([rxy[:, 0:16]] * N_HEADS, axis=-1)
    ry = jnp.concatenate([rxy[:, 16:32]] * N_HEADS, axis=-1)
    wvals = [w for (h, w) in SPATIAL]
    hvals = [h for (h, w) in SPATIAL]
    wf = _lane_const(wvals, jnp.float32)
    hf = _lane_const(hvals, jnp.float32)
    wi = _lane_const(wvals, jnp.int32)
    hi = _lane_const(hvals, jnp.int32)
    li = _lane_const(LSI, jnp.int32)
    hl = lax.broadcasted_iota(jnp.int32, (1, 128), 1) // 16
    x = rx * wf + sox - 0.5
    y = ry * hf + soy - 0.5
    x0f = jnp.floor(x)
    y0f = jnp.floor(y)
    fx = x - x0f
    fy = y - y0f
    x0 = x0f.astype(jnp.int32)
    y0 = y0f.astype(jnp.int32)
    x1 = x0 + 1
    y1 = y0 + 1
    vx0 = (x0 >= 0) & (x0 < wi)
    vx1 = (x1 >= 0) & (x1 < wi)
    vy0 = (y0 >= 0) & (y0 < hi)
    vy1 = (y1 >= 0) & (y1 < hi)
    cx0 = jnp.clip(x0, 0, wi - 1)
    cx1 = jnp.clip(x1, 0, wi - 1)
    cy0 = jnp.clip(y0, 0, hi - 1)
    cy1 = jnp.clip(y1, 0, hi - 1)
    base = li + fb * NQP
    r0 = base + cy0 * wi
    r1 = base + cy1 * wi
    idx_ref[0, 0] = (r0 + cx0) * 8 + hl
    idx_ref[0, 1] = (r0 + cx1) * 8 + hl
    idx_ref[0, 2] = (r1 + cx0) * 8 + hl
    idx_ref[0, 3] = (r1 + cx1) * 8 + hl
    gx0 = 1.0 - fx
    gy0 = 1.0 - fy
    wgt_ref[0, 0] = aw * gx0 * gy0 * (vx0 & vy0).astype(jnp.float32)
    wgt_ref[0, 1] = aw * fx * gy0 * (vx1 & vy0).astype(jnp.float32)
    wgt_ref[0, 2] = aw * gx0 * fy * (vx0 & vy1).astype(jnp.float32)
    wgt_ref[0, 3] = aw * fx * fy * (vx1 & vy1).astype(jnp.float32)


def _phase_a(src_pad, pos_pad, refxy, wcat, bcat, interpret=False):
    grid = (NFB, NA_CHUNKS)
    return pl.pallas_call(
        _a_body,
        grid=grid,
        in_specs=[
            pl.BlockSpec((1, 1, QC, D_MODEL),
                         lambda fb, qi: (fb // 2, fb % 2, qi, 0)),
            pl.BlockSpec((1, 1, QC, D_MODEL),
                         lambda fb, qi: (fb // 2, fb % 2, qi, 0)),
            pl.BlockSpec((1, 1, QC, 32),
                         lambda fb, qi: (fb // 2, fb % 2, qi, 0)),
            pl.BlockSpec((1, D_MODEL, NPROJ), lambda fb, qi: (fb // 2, 0, 0)),
            pl.BlockSpec((1, 8, NPROJ), lambda fb, qi: (fb // 2, 0, 0)),
        ],
        out_specs=[
            pl.BlockSpec((1, QC, D_MODEL), lambda fb, qi: (fb, qi, 0)),
            pl.BlockSpec((1, 4, QC, 128), lambda fb, qi: (fb, 0, qi, 0)),
            pl.BlockSpec((1, 4, QC, 128), lambda fb, qi: (fb, 0, qi, 0)),
        ],
        out_shape=[
            jax.ShapeDtypeStruct((NFB, NQP, D_MODEL), jnp.bfloat16),
            jax.ShapeDtypeStruct((NFB, 4, NQP, 128), jnp.int32),
            jax.ShapeDtypeStruct((NFB, 4, NQP, 128), jnp.float32),
        ],
        interpret=interpret,
    )(src_pad, pos_pad, refxy, wcat, bcat)


def _sc_gather_mac(vt_flat, idx4, wgt4):
    mesh = plsc.VectorSubcoreMesh(core_axis_name="c", subcore_axis_name="s")

    @functools.partial(
        pl.kernel,
        mesh=mesh,
        compiler_params=pltpu.CompilerParams(use_tc_tiling_on_sc=False,
                                             needs_layout_passes=False),
        out_type=jax.ShapeDtypeStruct((NFB, NQP, D_MODEL), jnp.float32),
        scratch_types=[
            pltpu.VMEM((GQ, 64), jnp.int32),
            pltpu.VMEM((GQ, 64), jnp.int32),
            pltpu.VMEM((GQ, 64), jnp.float32),
            pltpu.VMEM((GQ, 64), jnp.float32),
            pltpu.VMEM((CSC * 64, 32), jnp.bfloat16),
            pltpu.VMEM((CSC * 64, 32), jnp.bfloat16),
            pltpu.VMEM((GQ, 32), jnp.float32),
            pltpu.SemaphoreType.DMA,
            pltpu.SemaphoreType.DMA,
            pltpu.SemaphoreType.DMA,
            pltpu.SemaphoreType.DMA,
        ],
    )
    def k(vt_hbm, idx_hbm, wgt_hbm, out_hbm,
          ig0, ig1, wg0, wg1, r0, r1, outg, sg0, sg1, sr0, sr1):
        igs, wgs, rs = [ig0, ig1], [wg0, wg1], [r0, r1]
        sgs, srs = [sg0, sg1], [sr0, sr1]
        nc = plsc.get_sparse_core_info().num_cores
        wid = lax.axis_index("s") * nc + lax.axis_index("c")
        fb = wid // N_HEADS
        hd = wid % N_HEADS

        def group_copies(g, b, make_only):
            q0 = g * GQ
            f = pltpu.make_async_copy if make_only else pltpu.async_copy
            cps = []
            for c in range(4):
                cps.append(f(idx_hbm.at[fb, c, pl.ds(q0, GQ),
                                        pl.ds(hd * 16, 16)],
                             igs[b].at[:, pl.ds(c * 16, 16)], sgs[b]))
                cps.append(f(wgt_hbm.at[fb, c, pl.ds(q0, GQ),
                                        pl.ds(hd * 16, 16)],
                             wgs[b].at[:, pl.ds(c * 16, 16)], sgs[b]))
            return cps

        def fire(ig, ch, q):
            return [
                pltpu.async_copy(vt_hbm.at[ig.at[ch * CSC + lq]],
                                 rs[q].at[pl.ds(lq * 64, 64)], srs[q])
                for lq in range(CSC)
            ]

        def do_group(g, p):
            @pl.when(g + 1 < NG)
            def _():
                group_copies(g + 1, 1 - p, False)

            pending = fire(igs[p], 0, 0)
            for ch in range(GRP):
                q = ch % 2
                nxt = fire(igs[p], ch + 1, 1 - q) if ch + 1 < GRP else []
                for cp in pending:
                    cp.wait()
                pending = nxt

                @plsc.parallel_loop(0, CSC, 1, unroll=2)
                def q_body(qq):
                    acc0 = jnp.zeros((16,), jnp.float32)
                    acc1 = jnp.zeros((16,), jnp.float32)
                    for t in range(4):
                        wv = wgs[p][ch * CSC + qq, pl.ds(t * 16, 16)]
                        for e2 in range(16):
                            jx = qq * 64 + t * 16 + e2
                            wq = wv[e2]
                            lo, hi = plsc.unpack(
                                rs[q][jx, :],
                                format=plsc.PackFormat.INTERLEAVED)
                            acc0 = acc0 + wq * lo
                            acc1 = acc1 + wq * hi
                    outg[ch * CSC + qq, pl.ds(0, 16)] = acc0
                    outg[ch * CSC + qq, pl.ds(16, 16)] = acc1
            pltpu.sync_copy(outg,
                            out_hbm.at[fb, pl.ds(g * GQ, GQ),
                                       pl.ds(hd * 32, 32)])

        for cp in group_copies(0, 0, False):
            cp.wait()

        def pair_body(go, carry):
            for b in range(2):
                g = go * 2 + b

                @pl.when(g > 0)
                def _():
                    for cp in group_copies(g, b, True):
                        cp.wait()

                do_group(g, b)
            return carry

        lax.fori_loop(0, NG // 2, pair_body, 0)

    return k(vt_flat, idx4, wgt4)


def _finish_body(srcs_ref, pos_ref, attn_ref, ow_ref, ob_ref, lw_ref, lb_ref,
                 o_ref):
    src = srcs_ref[0, 0] + pos_ref[0, 0]
    y = src + lax.dot_general(
        attn_ref[0], ow_ref[0], (((1,), (1,)), ((), ())),
        preferred_element_type=jnp.float32) + ob_ref[0, 0:1, :]
    mu = jnp.mean(y, axis=-1, keepdims=True)
    var = jnp.mean((y - mu) ** 2, axis=-1, keepdims=True)
    o_ref[0] = (y - mu) * lax.rsqrt(var + 1e-5) * lw_ref[...] + lb_ref[...]


def _finish(src_pad, pos_pad, attn, ow_s, ob_s, ln_w, ln_b, interpret=False):
    grid = (NFB, NA_CHUNKS)
    return pl.pallas_call(
        _finish_body,
        grid=grid,
        in_specs=[
            pl.BlockSpec((1, 1, BQ, D_MODEL),
                         lambda fb, qi: (fb // 2, fb % 2, qi, 0)),
            pl.BlockSpec((1, 1, BQ, D_MODEL),
                         lambda fb, qi: (fb // 2, fb % 2, qi, 0)),
            pl.BlockSpec((1, BQ, D_MODEL), lambda fb, qi: (fb, qi, 0)),
            pl.BlockSpec((1, D_MODEL, D_MODEL), lambda fb, qi: (fb // 2, 0, 0)),
            pl.BlockSpec((1, 8, D_MODEL), lambda fb, qi: (fb // 2, 0, 0)),
            pl.BlockSpec((D_MODEL,), lambda fb, qi: (0,)),
            pl.BlockSpec((D_MODEL,), lambda fb, qi: (0,)),
        ],
        out_specs=pl.BlockSpec((1, BQ, D_MODEL), lambda fb, qi: (fb, qi, 0)),
        out_shape=jax.ShapeDtypeStruct((NFB, LEN_IN, D_MODEL), jnp.float32),
        interpret=interpret,
    )(src_pad, pos_pad, attn, ow_s, ob_s, ln_w, ln_b)


def _prep_weights(params):
    wcats, bcats, ows, obs = [], [], [], []
    for f in range(FEAT_NUM):
        p = params[f]
        sow = p["so_w"].reshape(N_HEADS, 16, 2, D_MODEL)
        vw = (p["value_w"].reshape(N_HEADS, 2, 16, D_MODEL)
              .transpose(0, 2, 1, 3).reshape(D_MODEL, D_MODEL))
        vb = (p["value_b"].reshape(N_HEADS, 2, 16)
              .transpose(0, 2, 1).reshape(D_MODEL))
        wc = jnp.concatenate([
            vw,
            sow[:, :, 0, :].reshape(128, D_MODEL),
            sow[:, :, 1, :].reshape(128, D_MODEL),
            p["aw_w"],
        ], axis=0)
        wcats.append(wc.T)
        sob = p["so_b"].reshape(N_HEADS, 16, 2)
        bc = jnp.concatenate([
            vb, sob[:, :, 0].reshape(128),
            sob[:, :, 1].reshape(128), p["aw_b"]], axis=0)
        bcats.append(jnp.broadcast_to(bc[None, :], (8, NPROJ)))
        ows.append(p["out_w"])
        obs.append(jnp.broadcast_to(p["out_b"][None, :], (8, D_MODEL)))
    wcat = jnp.stack(wcats).astype(jnp.bfloat16)
    bcat = jnp.stack(bcats)
    return wcat, bcat, jnp.stack(ows), jnp.stack(obs)


def kernel(srcs, pos, reference_points, spatial_shapes, level_start_index,
           padding_mask, params, ln_w, ln_b):
    del spatial_shapes, level_start_index, padding_mask
    pad_q = NQP - LEN_IN
    src_pad = jnp.pad(srcs, ((0, 0), (0, 0), (0, pad_q), (0, 0)))
    pos_pad = jnp.pad(pos, ((0, 0), (0, 0), (0, pad_q), (0, 0)))
    rx = jnp.repeat(reference_points[..., 0], N_POINTS, axis=-1)
    ry = jnp.repeat(reference_points[..., 1], N_POINTS, axis=-1)
    refxy = jnp.pad(jnp.concatenate([rx, ry], axis=-1),
                    ((0, 0), (0, 0), (0, pad_q), (0, 0)))
    wcat, bcat, ow_s, ob_s = _prep_weights(params)

    vt, idx4, wgt4 = _phase_a(src_pad, pos_pad, refxy, wcat, bcat)
    vt_flat = vt.reshape(NFB * NQP * 8, 32)

    attn = _sc_gather_mac(vt_flat, idx4, wgt4)

    out = _finish(src_pad, pos_pad, attn, ow_s, ob_s, ln_w, ln_b)
    return out.reshape(FEAT_NUM, BATCH, LEN_IN, D_MODEL)
